# Initial kernel scaffold; baseline (speedup 1.0000x reference)
import jax
import jax.numpy as jnp
from jax.experimental import pallas as pl


def kernel(x, table, W, b):
    raise NotImplementedError("write your pallas kernel here")


# speedup vs baseline: 13.3080x; 13.3080x over previous
import functools

import jax
import jax.numpy as jnp
from jax import lax
from jax.experimental import pallas as pl
from jax.experimental.pallas import tpu as pltpu
from jax.experimental.pallas import tpu_sc as plsc

_NC = 2
_NS = 16
_NW = _NC * _NS

_LANES = 16
_CHUNK = 128


def _tw_body(t_ref, w_ref, o_ref):
    o_ref[:] = jnp.sum(t_ref[:] * w_ref[0, :], axis=1)


def _compute_tw(table, W):
    V, D = table.shape
    R = 8192
    grid = (V + R - 1) // R
    return pl.pallas_call(
        _tw_body,
        grid=(grid,),
        in_specs=[
            pl.BlockSpec((R, D), lambda i: (i, 0)),
            pl.BlockSpec((1, D), lambda i: (0, 0)),
        ],
        out_specs=pl.BlockSpec((R,), lambda i: (i,)),
        out_shape=jax.ShapeDtypeStruct((V,), jnp.float32),
    )(table, W)


def _make_sc_kernel(B, L, V):
    rows_per_w = B // _NW
    n_chunks = rows_per_w // _CHUNK
    n_groups = _CHUNK // _LANES
    flat = L * _CHUNK

    mesh = plsc.VectorSubcoreMesh(core_axis_name="c", subcore_axis_name="s")

    @functools.partial(
        pl.kernel,
        mesh=mesh,
        out_type=jax.ShapeDtypeStruct((B,), jnp.float32),
        scratch_types=[
            pltpu.VMEM((flat,), jnp.int32),
            pltpu.VMEM((flat,), jnp.float32),
            pltpu.VMEM((rows_per_w,), jnp.float32),
            pltpu.VMEM((_LANES,), jnp.float32),
            pltpu.SemaphoreType.DMA,
        ],
    )
    def sc_kernel(xb_hbm, tw_hbm, b_hbm, out_hbm, idx_v, vals_v, out_v, b_v, sem):
        wid = lax.axis_index("s") * _NC + lax.axis_index("c")
        pltpu.sync_copy(b_hbm, b_v)
        bias = b_v[:]
        inv_l = jnp.float32(1.0 / L)

        def do_chunk(c, _):
            g = wid * n_chunks + c
            pltpu.sync_copy(xb_hbm.at[g], idx_v)
            pltpu.async_copy(tw_hbm.at[idx_v], vals_v, sem).wait()

            def accum(l, acc):
                base = l * _CHUNK
                return tuple(
                    acc[j] + vals_v[pl.ds(base + j * _LANES, _LANES)]
                    for j in range(n_groups)
                )

            zero = jnp.zeros((_LANES,), jnp.float32)
            acc = lax.fori_loop(0, L, accum, (zero,) * n_groups)
            for j in range(n_groups):
                z = acc[j] * inv_l + bias
                s = 1.0 / (1.0 + jnp.exp(-z))
                out_v[pl.ds(c * _CHUNK + j * _LANES, _LANES)] = s
            return 0

        lax.fori_loop(0, n_chunks, do_chunk, 0, unroll=True)
        pltpu.sync_copy(out_v, out_hbm.at[pl.ds(wid * rows_per_w, rows_per_w)])

    return sc_kernel


@jax.jit
def kernel(x, table, W, b):
    B, L = x.shape
    V, D = table.shape

    tw = _compute_tw(table, W)

    n_total_chunks = B // _CHUNK
    xb = (
        x.astype(jnp.int32)
        .reshape(n_total_chunks, _CHUNK, L)
        .transpose(0, 2, 1)
        .reshape(n_total_chunks, L * _CHUNK)
    )
    b_vec = jnp.broadcast_to(b.astype(jnp.float32), (_LANES,))

    out = _make_sc_kernel(B, L, V)(xb, tw, b_vec)
    return out.reshape(B, 1)
